# baseline (speedup 1.0000x reference)
import functools

import jax
import jax.numpy as jnp
from jax import lax
from jax.experimental import pallas as pl
from jax.experimental.pallas import tpu as pltpu
from jax.experimental.pallas import tpu_sc as plsc

NC = 2
NS = 16
NW = NC * NS
LANES = 16

N = 10000
D = 128
E = 320000
EPW = E // NW
CHUNK = 80
NFULL = EPW // CHUNK
RPT = N // NS

BLK = 2000
DEGR = N * LANES // 128


def _vsm():
    return plsc.VectorSubcoreMesh(core_axis_name="c", subcore_axis_name="s")


def _sc_degree(ei3):
    @functools.partial(
        pl.kernel,
        out_type=jax.ShapeDtypeStruct((NC, N, LANES), jnp.float32),
        mesh=_vsm(),
        compiler_params=pltpu.CompilerParams(use_tc_tiling_on_sc=False),
        scratch_types=[
            pltpu.VMEM((EPW,), jnp.int32),
            pltpu.VMEM((CHUNK, LANES), jnp.float32),
            pltpu.VMEM((RPT, LANES), jnp.float32),
            pltpu.VMEM_SHARED((N, LANES), jnp.float32),
            pltpu.SemaphoreType.DMA,
            pltpu.SemaphoreType.DMA,
        ],
    )
    def deg_kernel(ei_hbm, out_hbm, idx_v, ones_v, zbuf, acc, dsem, isem):
        cid = lax.axis_index("c")
        sid = lax.axis_index("s")
        wid = cid * NS + sid

        pltpu.async_copy(ei_hbm.at[pl.ds(E + wid * EPW, EPW)], idx_v, isem)

        def zfill_row(r, _):
            zbuf[r, :] = jnp.zeros((LANES,), jnp.float32)
            return 0

        lax.fori_loop(0, RPT, zfill_row, 0)

        pltpu.async_copy(zbuf, acc.at[pl.ds(sid * RPT, RPT)], dsem)

        def fill_row(r, _):
            ones_v[r, :] = jnp.ones((LANES,), jnp.float32)
            return 0

        lax.fori_loop(0, CHUNK, fill_row, 0)

        pltpu.make_async_copy(
            zbuf, acc.at[pl.ds(sid * RPT, RPT)], dsem).wait()
        pltpu.make_async_copy(
            ei_hbm.at[pl.ds(E + wid * EPW, EPW)], idx_v, isem).wait()
        plsc.subcore_barrier()

        def body(c, _):
            dst = acc.at[idx_v.at[pl.ds(c * CHUNK, CHUNK)]]
            pltpu.async_copy(ones_v, dst, dsem, add=True)

            @pl.when(c > 0)
            def _():
                prev = acc.at[idx_v.at[pl.ds((c - 1) * CHUNK, CHUNK)]]
                pltpu.make_async_copy(ones_v, prev, dsem).wait()
            return 0

        lax.fori_loop(0, NFULL, body, 0)
        last = acc.at[idx_v.at[pl.ds((NFULL - 1) * CHUNK, CHUNK)]]
        pltpu.make_async_copy(ones_v, last, dsem).wait()
        plsc.subcore_barrier()

        pltpu.sync_copy(
            acc.at[pl.ds(sid * RPT, RPT)],
            out_hbm.at[cid, pl.ds(sid * RPT, RPT)],
        )

    return deg_kernel(ei3)


def _sc_aggregate(hs, ei3):
    @functools.partial(
        pl.kernel,
        out_type=jax.ShapeDtypeStruct((NC, N, D), jnp.float32),
        mesh=_vsm(),
        compiler_params=pltpu.CompilerParams(use_tc_tiling_on_sc=False),
        scratch_types=[
            pltpu.VMEM((EPW,), jnp.int32),
            pltpu.VMEM((EPW,), jnp.int32),
            pltpu.VMEM((CHUNK, D), jnp.float32),
            pltpu.VMEM((CHUNK, D), jnp.float32),
            pltpu.VMEM((CHUNK, D), jnp.float32),
            pltpu.VMEM_SHARED((N, D), jnp.float32),
            pltpu.SemaphoreType.DMA,
            pltpu.SemaphoreType.DMA,
            pltpu.SemaphoreType.DMA,
            pltpu.SemaphoreType.DMA,
            pltpu.SemaphoreType.DMA,
            pltpu.SemaphoreType.DMA,
        ],
    )
    def agg_kernel(hs_hbm, ei_hbm, out_hbm,
                   src_v, dst_v, b0, b1, b2, acc,
                   g0, g1, g2, s0, s1, s2):
        cid = lax.axis_index("c")
        sid = lax.axis_index("s")
        wid = cid * NS + sid
        bufs = (b0, b1, b2)
        gsem = (g0, g1, g2)
        ssem = (s0, s1, s2)

        def zfill_row(r, _):
            for c8 in range(D // LANES):
                b0[r, pl.ds(c8 * LANES, LANES)] = jnp.zeros(
                    (LANES,), jnp.float32)
            return 0

        lax.fori_loop(0, CHUNK, zfill_row, 0)

        for z in range(RPT // CHUNK):
            pltpu.async_copy(
                b0, acc.at[pl.ds(sid * RPT + z * CHUNK, CHUNK)], s0)
        zrem = RPT % CHUNK
        if zrem:
            pltpu.async_copy(
                b0.at[pl.ds(0, zrem)],
                acc.at[pl.ds(sid * RPT + (RPT // CHUNK) * CHUNK, zrem)], s1)
        pltpu.async_copy(ei_hbm.at[pl.ds(wid * EPW, EPW)], src_v, g0)
        pltpu.async_copy(ei_hbm.at[pl.ds(E + wid * EPW, EPW)], dst_v, g1)
        for z in range(RPT // CHUNK):
            pltpu.make_async_copy(
                b0, acc.at[pl.ds(sid * RPT + z * CHUNK, CHUNK)], s0).wait()
        if zrem:
            pltpu.make_async_copy(
                b0.at[pl.ds(0, zrem)],
                acc.at[pl.ds(sid * RPT + (RPT // CHUNK) * CHUNK, zrem)],
                s1).wait()
        pltpu.make_async_copy(
            ei_hbm.at[pl.ds(wid * EPW, EPW)], src_v, g0).wait()
        pltpu.make_async_copy(
            ei_hbm.at[pl.ds(E + wid * EPW, EPW)], dst_v, g1).wait()
        plsc.subcore_barrier()

        def src_at(c):
            return src_v.at[pl.ds(c * CHUNK, CHUNK)]

        def dst_at(c):
            return dst_v.at[pl.ds(c * CHUNK, CHUNK)]

        def step(c, j):
            jn = (j + 1) % 3

            @pl.when(c + 1 < NFULL)
            def _():
                @pl.when(c + 1 >= 3)
                def _():
                    pltpu.make_async_copy(
                        bufs[jn], acc.at[dst_at(c - 2)], ssem[jn]).wait()
                pltpu.async_copy(hs_hbm.at[src_at(c + 1)], bufs[jn], gsem[jn])

            pltpu.make_async_copy(hs_hbm.at[src_at(c)], bufs[j],
                                  gsem[j]).wait()
            pltpu.async_copy(bufs[j], acc.at[dst_at(c)], ssem[j], add=True)

        pltpu.async_copy(hs_hbm.at[src_at(0)], b0, g0)

        def body(g, _):
            step(3 * g, 0)
            step(3 * g + 1, 1)
            step(3 * g + 2, 2)
            return 0

        NG = NFULL // 3
        lax.fori_loop(0, NG, body, 0)
        for c in range(3 * NG, NFULL):
            step(c, c % 3)
        for c in range(NFULL - 3, NFULL):
            j = c % 3
            pltpu.make_async_copy(bufs[j], acc.at[dst_at(c)], ssem[j]).wait()
        plsc.subcore_barrier()

        pltpu.sync_copy(
            acc.at[pl.ds(sid * RPT, RPT)],
            out_hbm.at[cid, pl.ds(sid * RPT, RPT)],
        )

    return agg_kernel(hs, ei3)


def _dinv_block(degp_ref):
    deg = degp_ref[0, :, 0] + degp_ref[1, :, 0] + 1.0
    return lax.rsqrt(deg)[:, None]


_DEG_SPEC = pl.BlockSpec((NC, BLK, LANES), lambda i: (0, i, 0))


def _tc_matmul_plain(x, W):
    def body(x_ref, w_ref, o_ref):
        o_ref[...] = jnp.dot(
            x_ref[...], w_ref[...], preferred_element_type=jnp.float32)

    return pl.pallas_call(
        body,
        grid=(N // BLK,),
        in_specs=[
            pl.BlockSpec((BLK, D), lambda i: (i, 0)),
            pl.BlockSpec((D, D), lambda i: (0, 0)),
        ],
        out_specs=pl.BlockSpec((BLK, D), lambda i: (i, 0)),
        out_shape=jax.ShapeDtypeStruct((N, D), jnp.float32),
    )(x, W)


def _tc_scale(degp, u):
    def body(degp_ref, u_ref, o_ref):
        o_ref[...] = _dinv_block(degp_ref) * u_ref[...]

    return pl.pallas_call(
        body,
        grid=(N // BLK,),
        in_specs=[
            _DEG_SPEC,
            pl.BlockSpec((BLK, D), lambda i: (i, 0)),
        ],
        out_specs=pl.BlockSpec((BLK, D), lambda i: (i, 0)),
        out_shape=jax.ShapeDtypeStruct((N, D), jnp.float32),
    )(degp, u)


def _tc_mid(degp, p, hs1, b1, W2):
    def body(degp_ref, p_ref, hs_ref, b_ref, w_ref, o_ref):
        dinv = _dinv_block(degp_ref)
        z = dinv * (p_ref[0] + p_ref[1] + hs_ref[...]) + b_ref[...]
        a = jnp.maximum(z, 0.0)
        o_ref[...] = dinv * jnp.dot(
            a, w_ref[...], preferred_element_type=jnp.float32)

    return pl.pallas_call(
        body,
        grid=(N // BLK,),
        in_specs=[
            _DEG_SPEC,
            pl.BlockSpec((NC, BLK, D), lambda i: (0, i, 0)),
            pl.BlockSpec((BLK, D), lambda i: (i, 0)),
            pl.BlockSpec((1, D), lambda i: (0, 0)),
            pl.BlockSpec((D, D), lambda i: (0, 0)),
        ],
        out_specs=pl.BlockSpec((BLK, D), lambda i: (i, 0)),
        out_shape=jax.ShapeDtypeStruct((N, D), jnp.float32),
    )(degp, p, hs1, b1, W2)


def _tc_final(degp, q, hs2, b2):
    def body(degp_ref, q_ref, hs_ref, b_ref, o_ref):
        dinv = _dinv_block(degp_ref)
        o_ref[...] = dinv * (q_ref[0] + q_ref[1] + hs_ref[...]) + b_ref[...]

    return pl.pallas_call(
        body,
        grid=(N // BLK,),
        in_specs=[
            _DEG_SPEC,
            pl.BlockSpec((NC, BLK, D), lambda i: (0, i, 0)),
            pl.BlockSpec((BLK, D), lambda i: (i, 0)),
            pl.BlockSpec((1, D), lambda i: (0, 0)),
        ],
        out_specs=pl.BlockSpec((BLK, D), lambda i: (i, 0)),
        out_shape=jax.ShapeDtypeStruct((N, D), jnp.float32),
    )(degp, q, hs2, b2)


def kernel(x, edge_index, W1, b1, W2, b2):
    ei3 = edge_index.astype(jnp.int32).reshape(2 * E)
    b1r = b1.reshape(1, D)
    b2r = b2.reshape(1, D)

    u1 = _tc_matmul_plain(x, W1)
    degp = _sc_degree(ei3)
    hs1 = _tc_scale(degp, u1)
    p = _sc_aggregate(hs1, ei3)
    hs2 = _tc_mid(degp, p, hs1, b1r, W2)
    q = _sc_aggregate(hs2, ei3)
    out = _tc_final(degp, q, hs2, b2r)
    return out
